# Initial kernel scaffold; baseline (speedup 1.0000x reference)
import jax
import jax.numpy as jnp
from jax.experimental import pallas as pl


def kernel(x, edge_index, batch, W1, b1, W2, b2, Wl, bl):
    raise NotImplementedError("write your pallas kernel here")


# speedup vs baseline: 9.0031x; 9.0031x over previous
import functools

import jax
import jax.numpy as jnp
from jax import lax
from jax.experimental import pallas as pl
from jax.experimental.pallas import tpu as pltpu
from jax.experimental.pallas import tpu_sc as plsc

_N = 10000
_E = 320000
_D = 128
_G = 128
_DOUT = 64

_NC, _NS, _L = 2, 16, 16
_NW = _NC * _NS
_K = 128
_CH = 80
_EPAD = _NW * _CH * _K
_NPAD = 10240
_RPT = _NPAD // _NS
_R = 1024
_NBLK = _NPAD // _R
_DUMMY = _NPAD - 1

@functools.cache
def _mesh():
    return plsc.VectorSubcoreMesh(
        core_axis_name="c", subcore_axis_name="s",
        num_cores=_NC, num_subcores=_NS)


def _sc_degree_body(dst_hbm, ones_hbm, zd_hbm, deg_hbm, dst_v, ones_v, acc):
    c = lax.axis_index("c")
    s = lax.axis_index("s")
    wid = s * _NC + c
    pltpu.sync_copy(zd_hbm, acc.at[pl.ds(s * _RPT, _RPT)])
    pltpu.sync_copy(ones_hbm, ones_v)
    pltpu.sync_copy(dst_hbm.at[wid], dst_v)
    plsc.subcore_barrier()

    def chunk(j, carry):
        pltpu.sync_copy(ones_v, acc.at[dst_v.at[j]], add=True)
        return carry

    lax.fori_loop(0, _CH, chunk, 0)
    plsc.subcore_barrier()
    pltpu.sync_copy(acc.at[pl.ds(s * _RPT, _RPT)],
                    deg_hbm.at[c, pl.ds(s * _RPT, _RPT)])


def _sc_degree(dst3, ones16, zd):
    return pl.kernel(
        _sc_degree_body,
        out_type=jax.ShapeDtypeStruct((_NC, _NPAD, _D), jnp.float32),
        mesh=_mesh(),
        scratch_types=[
            pltpu.VMEM((_CH, _K), jnp.int32),
            pltpu.VMEM((_K, _D), jnp.float32),
            pltpu.VMEM_SHARED((_NPAD, _D), jnp.float32),
        ],
    )(dst3, ones16, zd)


def _sc_scatter_body(u_hbm, src_hbm, dst_hbm, z2_hbm, out_hbm,
                     src_v, dst_v, rows_v, acc, sem):
    c = lax.axis_index("c")
    s = lax.axis_index("s")
    wid = s * _NC + c
    pltpu.sync_copy(z2_hbm, acc.at[pl.ds(s * _RPT, _RPT)])
    pltpu.sync_copy(src_hbm.at[wid], src_v)
    pltpu.sync_copy(dst_hbm.at[wid], dst_v)
    plsc.subcore_barrier()

    def chunk(j, carry):
        pltpu.async_copy(u_hbm.at[src_v.at[j]], rows_v, sem).wait()
        pltpu.sync_copy(rows_v, acc.at[dst_v.at[j]], add=True)
        return carry

    lax.fori_loop(0, _CH, chunk, 0)
    plsc.subcore_barrier()
    pltpu.sync_copy(acc.at[pl.ds(s * _RPT, _RPT)],
                    out_hbm.at[c, pl.ds(s * _RPT, _RPT)])


def _sc_scatter(u, src3, dst3, z2):
    return pl.kernel(
        _sc_scatter_body,
        out_type=jax.ShapeDtypeStruct((_NC, _NPAD, _D), jnp.float32),
        mesh=_mesh(),
        scratch_types=[
            pltpu.VMEM((_CH, _K), jnp.int32),
            pltpu.VMEM((_CH, _K), jnp.int32),
            pltpu.VMEM((_K, _D), jnp.float32),
            pltpu.VMEM_SHARED((_NPAD, _D), jnp.float32),
            pltpu.SemaphoreType.DMA,
        ],
    )(u, src3, dst3, z2)


def _dinv_of(degp_ref):
    sv = degp_ref[...]
    deg = sv[0, :, 0:1] + sv[1, :, 0:1] + 1.0
    return lax.rsqrt(deg)


def _tc_u1_body(degt_ref, x_ref, w_ref, o_ref):
    dinv = _dinv_of(degt_ref)
    o_ref[...] = dinv * jnp.dot(x_ref[...], w_ref[...],
                                preferred_element_type=jnp.float32)


def _tc_u1(degp, x_pad, W1):
    return pl.pallas_call(
        _tc_u1_body,
        grid=(_NBLK,),
        in_specs=[
            pl.BlockSpec((_NC, _R, _D), lambda i: (0, i, 0)),
            pl.BlockSpec((_R, _D), lambda i: (i, 0)),
            pl.BlockSpec((_D, _D), lambda i: (0, 0)),
        ],
        out_specs=pl.BlockSpec((_R, _D), lambda i: (i, 0)),
        out_shape=jax.ShapeDtypeStruct((_NPAD, _D), jnp.float32),
    )(degp, x_pad, W1)


def _tc_mid_body(degt_ref, s_ref, u_ref, b_ref, w_ref, o_ref):
    i = pl.program_id(0)
    dinv = _dinv_of(degt_ref)
    sv = s_ref[...]
    agg = dinv * (sv[0] + sv[1] + u_ref[...]) + b_ref[...]
    h = jnp.maximum(agg, 0.0)
    rows = lax.broadcasted_iota(jnp.int32, (_R, 1), 0) + i * _R
    h = jnp.where(rows < _N, h, 0.0)
    o_ref[...] = dinv * jnp.dot(h, w_ref[...],
                                preferred_element_type=jnp.float32)


def _tc_mid(degp, s1, u1, b1r, W2):
    return pl.pallas_call(
        _tc_mid_body,
        grid=(_NBLK,),
        in_specs=[
            pl.BlockSpec((_NC, _R, _D), lambda i: (0, i, 0)),
            pl.BlockSpec((_NC, _R, _D), lambda i: (0, i, 0)),
            pl.BlockSpec((_R, _D), lambda i: (i, 0)),
            pl.BlockSpec((1, _D), lambda i: (0, 0)),
            pl.BlockSpec((_D, _D), lambda i: (0, 0)),
        ],
        out_specs=pl.BlockSpec((_R, _D), lambda i: (i, 0)),
        out_shape=jax.ShapeDtypeStruct((_NPAD, _D), jnp.float32),
    )(degp, s1, u1, b1r, W2)


def _tc_final_body(degt_ref, s_ref, u_ref, b_ref, batch_ref, wl_ref, bl_ref,
                   o_ref, pool_acc, cnt_acc):
    i = pl.program_id(0)
    dinv = _dinv_of(degt_ref)
    sv = s_ref[...]
    h2 = dinv * (sv[0] + sv[1] + u_ref[...]) + b_ref[...]
    bt = batch_ref[0]
    gids = lax.broadcasted_iota(jnp.int32, (_G, _R), 0)
    oh = (gids == bt).astype(jnp.float32)

    @pl.when(i == 0)
    def _():
        pool_acc[...] = jnp.zeros_like(pool_acc)
        cnt_acc[...] = jnp.zeros_like(cnt_acc)

    pool_acc[...] += jnp.dot(oh, h2, preferred_element_type=jnp.float32)
    cnt_acc[...] += jnp.dot(oh, jnp.ones((_R, _D), jnp.float32),
                            preferred_element_type=jnp.float32)

    @pl.when(i == _NBLK - 1)
    def _():
        pooled = pool_acc[...] / jnp.maximum(cnt_acc[...], 1.0)
        o_ref[...] = jnp.maximum(
            jnp.dot(pooled, wl_ref[...],
                    preferred_element_type=jnp.float32) + bl_ref[...], 0.0)


def _tc_final(degp, s2, u2, b2r, batch3, Wl, blr):
    return pl.pallas_call(
        _tc_final_body,
        grid=(_NBLK,),
        in_specs=[
            pl.BlockSpec((_NC, _R, _D), lambda i: (0, i, 0)),
            pl.BlockSpec((_NC, _R, _D), lambda i: (0, i, 0)),
            pl.BlockSpec((_R, _D), lambda i: (i, 0)),
            pl.BlockSpec((1, _D), lambda i: (0, 0)),
            pl.BlockSpec((1, 1, _R), lambda i: (i, 0, 0)),
            pl.BlockSpec((_D, _DOUT), lambda i: (0, 0)),
            pl.BlockSpec((1, _DOUT), lambda i: (0, 0)),
        ],
        out_specs=pl.BlockSpec((_G, _DOUT), lambda i: (0, 0)),
        out_shape=jax.ShapeDtypeStruct((_G, _DOUT), jnp.float32),
        scratch_shapes=[
            pltpu.VMEM((_G, _D), jnp.float32),
            pltpu.VMEM((_G, _D), jnp.float32),
        ],
    )(degp, s2, u2, b2r, batch3, Wl, blr)


def kernel(x, edge_index, batch, W1, b1, W2, b2, Wl, bl):
    f32 = jnp.float32
    src = edge_index[0].astype(jnp.int32)
    dst = edge_index[1].astype(jnp.int32)
    pad_e = jnp.full((_EPAD - _E,), _DUMMY, jnp.int32)
    src3 = jnp.concatenate([src, pad_e]).reshape(_NW, _CH, _K)
    dst3 = jnp.concatenate([dst, pad_e]).reshape(_NW, _CH, _K)
    x_pad = jnp.zeros((_NPAD, _D), f32).at[:_N].set(x)
    batch3 = jnp.concatenate(
        [batch.astype(jnp.int32), jnp.full((_NPAD - _N,), -1, jnp.int32)]
    ).reshape(_NBLK, 1, _R)
    ones16 = jnp.ones((_K, _D), f32)
    z2 = jnp.zeros((_RPT, _D), f32)
    zd = z2
    b1r = b1.reshape(1, _D)
    b2r = b2.reshape(1, _D)
    blr = bl.reshape(1, _DOUT)

    degp = _sc_degree(dst3, ones16, zd)
    u1 = _tc_u1(degp, x_pad, W1)
    s1 = _sc_scatter(u1, src3, dst3, z2)
    u2 = _tc_mid(degp, s1, u1, b1r, W2)
    s2 = _sc_scatter(u2, src3, dst3, z2)
    out = _tc_final(degp, s2, u2, b2r, batch3, Wl, blr)
    return out


# speedup vs baseline: 10.0317x; 1.1142x over previous
import functools

import jax
import jax.numpy as jnp
from jax import lax
from jax.experimental import pallas as pl
from jax.experimental.pallas import tpu as pltpu
from jax.experimental.pallas import tpu_sc as plsc

_N = 10000
_E = 320000
_D = 128
_G = 128
_DOUT = 64

_NC, _NS, _L = 2, 16, 16
_NW = _NC * _NS
_K = 128
_CH = 80
_EPAD = _NW * _CH * _K
_NPAD = 10240
_RPT = _NPAD // _NS
_R = 1024
_NBLK = _NPAD // _R
_DUMMY = _NPAD - 1

@functools.cache
def _mesh():
    return plsc.VectorSubcoreMesh(
        core_axis_name="c", subcore_axis_name="s",
        num_cores=_NC, num_subcores=_NS)


_NB = 2

def _sc_degree_body(dst_hbm, ones_hbm, zd_hbm, deg_hbm, dst_v, ones_v, acc):
    c = lax.axis_index("c")
    s = lax.axis_index("s")
    wid = s * _NC + c
    pltpu.sync_copy(zd_hbm, acc.at[pl.ds(s * _RPT, _RPT)])
    pltpu.sync_copy(ones_hbm, ones_v)
    pltpu.sync_copy(dst_hbm.at[wid], dst_v)
    plsc.subcore_barrier()

    def chunk(j, carry):
        pltpu.sync_copy(ones_v, acc.at[dst_v.at[j]], add=True)
        return carry

    lax.fori_loop(0, _CH, chunk, 0)
    plsc.subcore_barrier()
    pltpu.sync_copy(acc.at[pl.ds(s * _RPT, _RPT)],
                    deg_hbm.at[c, pl.ds(s * _RPT, _RPT)])


def _sc_degree(dst3, ones16, zd):
    return pl.kernel(
        _sc_degree_body,
        out_type=jax.ShapeDtypeStruct((_NC, _NPAD, _D), jnp.float32),
        mesh=_mesh(),
        scratch_types=(
            [pltpu.VMEM((_CH, _K), jnp.int32),
             pltpu.VMEM((_K, _D), jnp.float32),
             pltpu.VMEM_SHARED((_NPAD, _D), jnp.float32)]
        ),
    )(dst3, ones16, zd)


def _sc_scatter_body(u_hbm, src_hbm, dst_hbm, z2_hbm, out_hbm, *rest):
    srcb = rest[0:2]
    rows = rest[2:4]
    dst_v = rest[4]
    acc = rest[5]
    isem = rest[6:8]
    gsem = rest[8:10]
    c = lax.axis_index("c")
    s = lax.axis_index("s")
    wid = s * _NC + c
    pltpu.sync_copy(z2_hbm, acc.at[pl.ds(s * _RPT, _RPT)])
    pltpu.sync_copy(dst_hbm.at[wid], dst_v)
    for b in range(2):
        pltpu.async_copy(src_hbm.at[wid, pl.ds(b, 1)], srcb[b], isem[b])
    plsc.subcore_barrier()

    def _iwait(b):
        pltpu.make_async_copy(src_hbm.at[wid, pl.ds(0, 1)], srcb[b],
                              isem[b]).wait()

    def _gwait(b):
        pltpu.make_async_copy(u_hbm.at[pl.ds(0, _K)], rows[b],
                              gsem[b]).wait()

    _iwait(0)
    pltpu.async_copy(u_hbm.at[srcb[0].at[0]], rows[0], gsem[0])

    def mega(m, carry):
        j0 = m * 2
        for b in range(2):
            bn = 1 - b
            _iwait(bn)
            pltpu.async_copy(u_hbm.at[srcb[bn].at[0]], rows[bn], gsem[bn])
            _gwait(b)
            pltpu.sync_copy(rows[b], acc.at[dst_v.at[j0 + b]], add=True)
            jf = jnp.minimum(j0 + b + 2, _CH - 1)
            pltpu.async_copy(src_hbm.at[wid, pl.ds(jf, 1)], srcb[b], isem[b])
        return carry

    lax.fori_loop(0, _CH // 2, mega, 0)
    _gwait(0)
    _iwait(1)
    plsc.subcore_barrier()
    pltpu.sync_copy(acc.at[pl.ds(s * _RPT, _RPT)],
                    out_hbm.at[c, pl.ds(s * _RPT, _RPT)])


def _sc_scatter(u, src3, dst3, z2):
    return pl.kernel(
        _sc_scatter_body,
        out_type=jax.ShapeDtypeStruct((_NC, _NPAD, _D), jnp.float32),
        mesh=_mesh(),
        scratch_types=(
            [pltpu.VMEM((1, _K), jnp.int32)] * 2
            + [pltpu.VMEM((_K, _D), jnp.float32)] * 2
            + [pltpu.VMEM((_CH, _K), jnp.int32)]
            + [pltpu.VMEM_SHARED((_NPAD, _D), jnp.float32)]
            + [pltpu.SemaphoreType.DMA] * 4
        ),
    )(u, src3, dst3, z2)


def _dinv_of(degp_ref):
    sv = degp_ref[...]
    deg = sv[0, :, 0:1] + sv[1, :, 0:1] + 1.0
    return lax.rsqrt(deg)


def _tc_u1_body(degt_ref, x_ref, w_ref, o_ref):
    dinv = _dinv_of(degt_ref)
    o_ref[...] = dinv * jnp.dot(x_ref[...], w_ref[...],
                                preferred_element_type=jnp.float32)


def _tc_u1(degp, x_pad, W1):
    return pl.pallas_call(
        _tc_u1_body,
        grid=(_NBLK,),
        in_specs=[
            pl.BlockSpec((_NC, _R, _D), lambda i: (0, i, 0)),
            pl.BlockSpec((_R, _D), lambda i: (i, 0)),
            pl.BlockSpec((_D, _D), lambda i: (0, 0)),
        ],
        out_specs=pl.BlockSpec((_R, _D), lambda i: (i, 0)),
        out_shape=jax.ShapeDtypeStruct((_NPAD, _D), jnp.float32),
    )(degp, x_pad, W1)


def _tc_mid_body(degt_ref, s_ref, u_ref, b_ref, w_ref, o_ref):
    i = pl.program_id(0)
    dinv = _dinv_of(degt_ref)
    sv = s_ref[...]
    agg = dinv * (sv[0] + sv[1] + u_ref[...]) + b_ref[...]
    h = jnp.maximum(agg, 0.0)
    rows = lax.broadcasted_iota(jnp.int32, (_R, 1), 0) + i * _R
    h = jnp.where(rows < _N, h, 0.0)
    o_ref[...] = dinv * jnp.dot(h, w_ref[...],
                                preferred_element_type=jnp.float32)


def _tc_mid(degp, s1, u1, b1r, W2):
    return pl.pallas_call(
        _tc_mid_body,
        grid=(_NBLK,),
        in_specs=[
            pl.BlockSpec((_NC, _R, _D), lambda i: (0, i, 0)),
            pl.BlockSpec((_NC, _R, _D), lambda i: (0, i, 0)),
            pl.BlockSpec((_R, _D), lambda i: (i, 0)),
            pl.BlockSpec((1, _D), lambda i: (0, 0)),
            pl.BlockSpec((_D, _D), lambda i: (0, 0)),
        ],
        out_specs=pl.BlockSpec((_R, _D), lambda i: (i, 0)),
        out_shape=jax.ShapeDtypeStruct((_NPAD, _D), jnp.float32),
    )(degp, s1, u1, b1r, W2)


def _tc_final_body(degt_ref, s_ref, u_ref, b_ref, batch_ref, wl_ref, bl_ref,
                   o_ref, pool_acc, cnt_acc):
    i = pl.program_id(0)
    dinv = _dinv_of(degt_ref)
    sv = s_ref[...]
    h2 = dinv * (sv[0] + sv[1] + u_ref[...]) + b_ref[...]
    bt = batch_ref[0]
    gids = lax.broadcasted_iota(jnp.int32, (_G, _R), 0)
    oh = (gids == bt).astype(jnp.float32)

    @pl.when(i == 0)
    def _():
        pool_acc[...] = jnp.zeros_like(pool_acc)
        cnt_acc[...] = jnp.zeros_like(cnt_acc)

    pool_acc[...] += jnp.dot(oh, h2, preferred_element_type=jnp.float32)
    cnt_acc[...] += jnp.dot(oh, jnp.ones((_R, _D), jnp.float32),
                            preferred_element_type=jnp.float32)

    @pl.when(i == _NBLK - 1)
    def _():
        pooled = pool_acc[...] / jnp.maximum(cnt_acc[...], 1.0)
        o_ref[...] = jnp.maximum(
            jnp.dot(pooled, wl_ref[...],
                    preferred_element_type=jnp.float32) + bl_ref[...], 0.0)


def _tc_final(degp, s2, u2, b2r, batch3, Wl, blr):
    return pl.pallas_call(
        _tc_final_body,
        grid=(_NBLK,),
        in_specs=[
            pl.BlockSpec((_NC, _R, _D), lambda i: (0, i, 0)),
            pl.BlockSpec((_NC, _R, _D), lambda i: (0, i, 0)),
            pl.BlockSpec((_R, _D), lambda i: (i, 0)),
            pl.BlockSpec((1, _D), lambda i: (0, 0)),
            pl.BlockSpec((1, 1, _R), lambda i: (i, 0, 0)),
            pl.BlockSpec((_D, _DOUT), lambda i: (0, 0)),
            pl.BlockSpec((1, _DOUT), lambda i: (0, 0)),
        ],
        out_specs=pl.BlockSpec((_G, _DOUT), lambda i: (0, 0)),
        out_shape=jax.ShapeDtypeStruct((_G, _DOUT), jnp.float32),
        scratch_shapes=[
            pltpu.VMEM((_G, _D), jnp.float32),
            pltpu.VMEM((_G, _D), jnp.float32),
        ],
    )(degp, s2, u2, b2r, batch3, Wl, blr)


def kernel(x, edge_index, batch, W1, b1, W2, b2, Wl, bl):
    f32 = jnp.float32
    src = edge_index[0].astype(jnp.int32)
    dst = edge_index[1].astype(jnp.int32)
    pad_e = jnp.full((_EPAD - _E,), _DUMMY, jnp.int32)
    src3 = jnp.concatenate([src, pad_e]).reshape(_NW, _CH, _K)
    dst3 = jnp.concatenate([dst, pad_e]).reshape(_NW, _CH, _K)
    x_pad = jnp.zeros((_NPAD, _D), f32).at[:_N].set(x)
    batch3 = jnp.concatenate(
        [batch.astype(jnp.int32), jnp.full((_NPAD - _N,), -1, jnp.int32)]
    ).reshape(_NBLK, 1, _R)
    ones16 = jnp.ones((_K, _D), f32)
    z2 = jnp.zeros((_RPT, _D), f32)
    zd = z2
    b1r = b1.reshape(1, _D)
    b2r = b2.reshape(1, _D)
    blr = bl.reshape(1, _DOUT)

    degp = _sc_degree(dst3, ones16, zd)
    u1 = _tc_u1(degp, x_pad, W1)
    s1 = _sc_scatter(u1, src3, dst3, z2)
    u2 = _tc_mid(degp, s1, u1, b1r, W2)
    s2 = _sc_scatter(u2, src3, dst3, z2)
    out = _tc_final(degp, s2, u2, b2r, batch3, Wl, blr)
    return out


# speedup vs baseline: 10.5913x; 1.0558x over previous
import functools

import jax
import jax.numpy as jnp
from jax import lax
from jax.experimental import pallas as pl
from jax.experimental.pallas import tpu as pltpu
from jax.experimental.pallas import tpu_sc as plsc

_N = 10000
_E = 320000
_D = 128
_G = 128
_DOUT = 64

_NC, _NS, _L = 2, 16, 16
_NW = _NC * _NS
_K = 128
_CH = 80
_EPAD = _NW * _CH * _K
_NPAD = 10240
_RPT = _NPAD // _NS
_R = 1024
_NBLK = _NPAD // _R
_DUMMY = _NPAD - 1

@functools.cache
def _mesh():
    return plsc.VectorSubcoreMesh(
        core_axis_name="c", subcore_axis_name="s",
        num_cores=_NC, num_subcores=_NS)


_NB = 2

def _sc_degree_body(dst_hbm, ones_hbm, zd_hbm, deg_hbm, dst_v, ones_v, acc):
    c = lax.axis_index("c")
    s = lax.axis_index("s")
    wid = s * _NC + c
    pltpu.sync_copy(zd_hbm, acc.at[pl.ds(s * _RPT, _RPT)])
    pltpu.sync_copy(ones_hbm, ones_v)
    pltpu.sync_copy(dst_hbm.at[wid], dst_v)
    plsc.subcore_barrier()

    def chunk(j, carry):
        pltpu.sync_copy(ones_v, acc.at[dst_v.at[j]], add=True)
        return carry

    lax.fori_loop(0, _CH, chunk, 0)
    plsc.subcore_barrier()
    pltpu.sync_copy(acc.at[pl.ds(s * _RPT, _RPT)],
                    deg_hbm.at[c, pl.ds(s * _RPT, _RPT)])


def _sc_degree(dst3, ones16, zd):
    return pl.kernel(
        _sc_degree_body,
        out_type=jax.ShapeDtypeStruct((_NC, _NPAD, _D), jnp.float32),
        mesh=_mesh(),
        scratch_types=(
            [pltpu.VMEM((_CH, _K), jnp.int32),
             pltpu.VMEM((_K, _D), jnp.float32),
             pltpu.VMEM_SHARED((_NPAD, _D), jnp.float32)]
        ),
    )(dst3, ones16, zd)


def _sc_scatter_body(u_hbm, src_hbm, dst_hbm, z2_hbm, out_hbm, *rest):
    srcb = rest[0:2]
    rows = rest[2:4]
    dst_v = rest[4]
    acc = rest[5]
    isem = rest[6:8]
    gsem = rest[8:10]
    c = lax.axis_index("c")
    s = lax.axis_index("s")
    wid = s * _NC + c
    pltpu.sync_copy(z2_hbm, acc.at[pl.ds(s * _RPT, _RPT)])
    pltpu.sync_copy(dst_hbm.at[wid], dst_v)
    for b in range(2):
        pltpu.async_copy(src_hbm.at[wid, pl.ds(b, 1)], srcb[b], isem[b])
    plsc.subcore_barrier()

    def _iwait(b):
        pltpu.make_async_copy(src_hbm.at[wid, pl.ds(0, 1)], srcb[b],
                              isem[b]).wait()

    def _gwait(b):
        pltpu.make_async_copy(u_hbm.at[pl.ds(0, _K)], rows[b],
                              gsem[b]).wait()

    _iwait(0)
    pltpu.async_copy(u_hbm.at[srcb[0].at[0]], rows[0], gsem[0])

    def mega(m, carry):
        j0 = m * 2
        for b in range(2):
            bn = 1 - b
            _iwait(bn)
            pltpu.async_copy(u_hbm.at[srcb[bn].at[0]], rows[bn], gsem[bn])
            _gwait(b)
            pltpu.sync_copy(rows[b], acc.at[dst_v.at[j0 + b]], add=True)
            jf = jnp.minimum(j0 + b + 2, _CH - 1)
            pltpu.async_copy(src_hbm.at[wid, pl.ds(jf, 1)], srcb[b], isem[b])
        return carry

    lax.fori_loop(0, _CH // 2, mega, 0)
    _gwait(0)
    _iwait(1)
    plsc.subcore_barrier()
    pltpu.sync_copy(acc.at[pl.ds(s * _RPT, _RPT)],
                    out_hbm.at[c, pl.ds(s * _RPT, _RPT)])


def _sc_scatter(u, src3, dst3, z2):
    return pl.kernel(
        _sc_scatter_body,
        out_type=jax.ShapeDtypeStruct((_NC, _NPAD, _D), jnp.float32),
        mesh=_mesh(),
        scratch_types=(
            [pltpu.VMEM((1, _K), jnp.int32)] * 2
            + [pltpu.VMEM((_K, _D), jnp.float32)] * 2
            + [pltpu.VMEM((_CH, _K), jnp.int32)]
            + [pltpu.VMEM_SHARED((_NPAD, _D), jnp.float32)]
            + [pltpu.SemaphoreType.DMA] * 4
        ),
    )(u, src3, dst3, z2)


def _dinv_of(degp_ref):
    sv = degp_ref[...]
    deg = sv[0, :, 0:1] + sv[1, :, 0:1] + 1.0
    return lax.rsqrt(deg)


def _tc_h1_body(x_ref, w_ref, o_ref):
    o_ref[...] = jnp.dot(x_ref[...], w_ref[...],
                         preferred_element_type=jnp.float32)


def _tc_h1(x_pad, W1):
    return pl.pallas_call(
        _tc_h1_body,
        grid=(_NBLK,),
        in_specs=[
            pl.BlockSpec((_R, _D), lambda i: (i, 0)),
            pl.BlockSpec((_D, _D), lambda i: (0, 0)),
        ],
        out_specs=pl.BlockSpec((_R, _D), lambda i: (i, 0)),
        out_shape=jax.ShapeDtypeStruct((_NPAD, _D), jnp.float32),
    )(x_pad, W1)


def _tc_u1_body(degt_ref, h_ref, o_ref):
    dinv = _dinv_of(degt_ref)
    o_ref[...] = dinv * h_ref[...]


def _tc_u1(degp, h1):
    return pl.pallas_call(
        _tc_u1_body,
        grid=(_NBLK,),
        in_specs=[
            pl.BlockSpec((_NC, _R, _D), lambda i: (0, i, 0)),
            pl.BlockSpec((_R, _D), lambda i: (i, 0)),
        ],
        out_specs=pl.BlockSpec((_R, _D), lambda i: (i, 0)),
        out_shape=jax.ShapeDtypeStruct((_NPAD, _D), jnp.float32),
    )(degp, h1)


def _tc_mid_body(degt_ref, s_ref, u_ref, b_ref, w_ref, o_ref):
    i = pl.program_id(0)
    dinv = _dinv_of(degt_ref)
    sv = s_ref[...]
    agg = dinv * (sv[0] + sv[1] + u_ref[...]) + b_ref[...]
    h = jnp.maximum(agg, 0.0)
    rows = lax.broadcasted_iota(jnp.int32, (_R, 1), 0) + i * _R
    h = jnp.where(rows < _N, h, 0.0)
    o_ref[...] = dinv * jnp.dot(h, w_ref[...],
                                preferred_element_type=jnp.float32)


def _tc_mid(degp, s1, u1, b1r, W2):
    return pl.pallas_call(
        _tc_mid_body,
        grid=(_NBLK,),
        in_specs=[
            pl.BlockSpec((_NC, _R, _D), lambda i: (0, i, 0)),
            pl.BlockSpec((_NC, _R, _D), lambda i: (0, i, 0)),
            pl.BlockSpec((_R, _D), lambda i: (i, 0)),
            pl.BlockSpec((1, _D), lambda i: (0, 0)),
            pl.BlockSpec((_D, _D), lambda i: (0, 0)),
        ],
        out_specs=pl.BlockSpec((_R, _D), lambda i: (i, 0)),
        out_shape=jax.ShapeDtypeStruct((_NPAD, _D), jnp.float32),
    )(degp, s1, u1, b1r, W2)


def _tc_final_body(degt_ref, s_ref, u_ref, b_ref, batch_ref, wl_ref, bl_ref,
                   o_ref, pool_acc, cnt_acc):
    i = pl.program_id(0)
    dinv = _dinv_of(degt_ref)
    sv = s_ref[...]
    h2 = dinv * (sv[0] + sv[1] + u_ref[...]) + b_ref[...]
    bt = batch_ref[0]
    gids = lax.broadcasted_iota(jnp.int32, (_G, _R), 0)
    oh = (gids == bt).astype(jnp.float32)

    @pl.when(i == 0)
    def _():
        pool_acc[...] = jnp.zeros_like(pool_acc)
        cnt_acc[...] = jnp.zeros_like(cnt_acc)

    pool_acc[...] += jnp.dot(oh, h2, preferred_element_type=jnp.float32)
    cnt_acc[...] += jnp.dot(oh, jnp.ones((_R, _D), jnp.float32),
                            preferred_element_type=jnp.float32)

    @pl.when(i == _NBLK - 1)
    def _():
        pooled = pool_acc[...] / jnp.maximum(cnt_acc[...], 1.0)
        o_ref[...] = jnp.maximum(
            jnp.dot(pooled, wl_ref[...],
                    preferred_element_type=jnp.float32) + bl_ref[...], 0.0)


def _tc_final(degp, s2, u2, b2r, batch3, Wl, blr):
    return pl.pallas_call(
        _tc_final_body,
        grid=(_NBLK,),
        in_specs=[
            pl.BlockSpec((_NC, _R, _D), lambda i: (0, i, 0)),
            pl.BlockSpec((_NC, _R, _D), lambda i: (0, i, 0)),
            pl.BlockSpec((_R, _D), lambda i: (i, 0)),
            pl.BlockSpec((1, _D), lambda i: (0, 0)),
            pl.BlockSpec((1, 1, _R), lambda i: (i, 0, 0)),
            pl.BlockSpec((_D, _DOUT), lambda i: (0, 0)),
            pl.BlockSpec((1, _DOUT), lambda i: (0, 0)),
        ],
        out_specs=pl.BlockSpec((_G, _DOUT), lambda i: (0, 0)),
        out_shape=jax.ShapeDtypeStruct((_G, _DOUT), jnp.float32),
        scratch_shapes=[
            pltpu.VMEM((_G, _D), jnp.float32),
            pltpu.VMEM((_G, _D), jnp.float32),
        ],
    )(degp, s2, u2, b2r, batch3, Wl, blr)


def kernel(x, edge_index, batch, W1, b1, W2, b2, Wl, bl):
    f32 = jnp.float32
    src = edge_index[0].astype(jnp.int32)
    dst = edge_index[1].astype(jnp.int32)
    pad_e = jnp.full((_EPAD - _E,), _DUMMY, jnp.int32)
    src3 = jnp.concatenate([src, pad_e]).reshape(_NW, _CH, _K)
    dst3 = jnp.concatenate([dst, pad_e]).reshape(_NW, _CH, _K)
    x_pad = jnp.zeros((_NPAD, _D), f32).at[:_N].set(x)
    batch3 = jnp.concatenate(
        [batch.astype(jnp.int32), jnp.full((_NPAD - _N,), -1, jnp.int32)]
    ).reshape(_NBLK, 1, _R)
    ones16 = jnp.ones((_K, _D), f32)
    z2 = jnp.zeros((_RPT, _D), f32)
    zd = z2
    b1r = b1.reshape(1, _D)
    b2r = b2.reshape(1, _D)
    blr = bl.reshape(1, _DOUT)

    degp = _sc_degree(dst3, ones16, zd)
    h1 = _tc_h1(x_pad, W1)
    u1 = _tc_u1(degp, h1)
    s1 = _sc_scatter(u1, src3, dst3, z2)
    u2 = _tc_mid(degp, s1, u1, b1r, W2)
    s2 = _sc_scatter(u2, src3, dst3, z2)
    out = _tc_final(degp, s2, u2, b2r, batch3, Wl, blr)
    return out
